# baseline (speedup 1.0000x reference)
import functools

import jax
import jax.numpy as jnp
from jax import lax
from jax.experimental import pallas as pl
from jax.experimental.pallas import tpu as pltpu
from jax.experimental.pallas import tpu_sc as plsc

N_CLASSES = 100000
EMBED_DIM = 64
BATCH = 16384

_info = plsc.get_sparse_core_info()
_NC, _NS = _info.num_cores, _info.num_subcores
_NW = _NC * _NS
_B_PER_W = BATCH // _NW
_CHUNK = 128
_NCHUNK = _B_PER_W // _CHUNK


@functools.partial(
    pl.kernel,
    mesh=plsc.VectorSubcoreMesh(core_axis_name="c", subcore_axis_name="s"),
    out_type=jax.ShapeDtypeStruct((BATCH, EMBED_DIM), jnp.float32),
    scratch_types=[
        pltpu.VMEM((_B_PER_W,), jnp.int32),
        [pltpu.VMEM((_CHUNK, EMBED_DIM), jnp.float32) for _ in range(2)],
        [pltpu.SemaphoreType.DMA for _ in range(2)],
        pltpu.SemaphoreType.DMA,
    ],
    compiler_params=pltpu.CompilerParams(use_tc_tiling_on_sc=False),
)
def _gather_kernel(idx_hbm, table_hbm, out_hbm, idx_v, rows, gsems, ssem):
    wid = lax.axis_index("s") * _NC + lax.axis_index("c")
    base = wid * _B_PER_W
    pltpu.sync_copy(idx_hbm.at[pl.ds(base, _B_PER_W)], idx_v)

    def fire_chunk(c, buf, sem):
        def body(j, _):
            v = idx_v[pl.ds(c * _CHUNK + j * 16, 16)]
            for k in range(16):
                pltpu.async_copy(
                    table_hbm.at[pl.ds(v[k], 1)],
                    buf.at[pl.ds(j * 16 + k, 1)],
                    sem,
                )
            return 0

        lax.fori_loop(0, _CHUNK // 16, body, 0)

    def drain_chunk(buf, sem):
        pltpu.make_async_copy(
            table_hbm.at[pl.ds(0, _CHUNK)], buf, sem
        ).wait()

    fire_chunk(0, rows[0], gsems[0])
    scatters = [None, None]
    for c in range(_NCHUNK):
        if c + 1 < _NCHUNK:
            if scatters[(c + 1) % 2] is not None:
                scatters[(c + 1) % 2].wait()
                scatters[(c + 1) % 2] = None
            fire_chunk(c + 1, rows[(c + 1) % 2], gsems[(c + 1) % 2])
        drain_chunk(rows[c % 2], gsems[c % 2])
        scatters[c % 2] = pltpu.async_copy(
            rows[c % 2], out_hbm.at[pl.ds(base + c * _CHUNK, _CHUNK)], ssem
        )
    for s in scatters:
        if s is not None:
            s.wait()


def kernel(x, table):
    return _gather_kernel(x.astype(jnp.int32), table)


# speedup vs baseline: 1.4820x; 1.4820x over previous
import functools

import jax
import jax.numpy as jnp
from jax import lax
from jax.experimental import pallas as pl
from jax.experimental.pallas import tpu as pltpu
from jax.experimental.pallas import tpu_sc as plsc

N_CLASSES = 100000
EMBED_DIM = 64
BATCH = 16384

_info = plsc.get_sparse_core_info()
_NC, _NS = _info.num_cores, _info.num_subcores
_NW = _NC * _NS
_B_PER_W = BATCH // _NW
_CHUNK = 128
_NCHUNK = _B_PER_W // _CHUNK


@functools.partial(
    pl.kernel,
    mesh=plsc.VectorSubcoreMesh(core_axis_name="c", subcore_axis_name="s"),
    out_type=jax.ShapeDtypeStruct((BATCH, EMBED_DIM), jnp.float32),
    scratch_types=[
        pltpu.VMEM((_B_PER_W,), jnp.int32),
        [pltpu.VMEM((_CHUNK, EMBED_DIM), jnp.float32) for _ in range(2)],
        [pltpu.SemaphoreType.DMA for _ in range(2)],
        pltpu.SemaphoreType.DMA,
    ],
    compiler_params=pltpu.CompilerParams(use_tc_tiling_on_sc=True),
)
def _gather_kernel(idx_hbm, table_hbm, out_hbm, idx_v, rows, gsems, ssem):
    wid = lax.axis_index("s") * _NC + lax.axis_index("c")
    base = wid * _B_PER_W
    pltpu.sync_copy(idx_hbm.at[pl.ds(base, _B_PER_W)], idx_v)

    def fire_chunk(c, buf, sem):
        def body(j, _):
            v = idx_v[pl.ds(c * _CHUNK + j * 16, 16)]
            for k in range(16):
                pltpu.async_copy(
                    table_hbm.at[pl.ds(v[k], 1)],
                    buf.at[pl.ds(j * 16 + k, 1)],
                    sem,
                )
            return 0

        lax.fori_loop(0, _CHUNK // 16, body, 0)

    def drain_chunk(buf, sem):
        pltpu.make_async_copy(
            table_hbm.at[pl.ds(0, _CHUNK)], buf, sem
        ).wait()

    fire_chunk(0, rows[0], gsems[0])
    scatters = [None, None]
    for c in range(_NCHUNK):
        if c + 1 < _NCHUNK:
            if scatters[(c + 1) % 2] is not None:
                scatters[(c + 1) % 2].wait()
                scatters[(c + 1) % 2] = None
            fire_chunk(c + 1, rows[(c + 1) % 2], gsems[(c + 1) % 2])
        drain_chunk(rows[c % 2], gsems[c % 2])
        scatters[c % 2] = pltpu.async_copy(
            rows[c % 2], out_hbm.at[pl.ds(base + c * _CHUNK, _CHUNK)], ssem
        )
    for s in scatters:
        if s is not None:
            s.wait()


def kernel(x, table):
    return _gather_kernel(x.astype(jnp.int32), table)


# speedup vs baseline: 1.4941x; 1.0081x over previous
import functools

import jax
import jax.numpy as jnp
from jax import lax
from jax.experimental import pallas as pl
from jax.experimental.pallas import tpu as pltpu
from jax.experimental.pallas import tpu_sc as plsc

N_CLASSES = 100000
EMBED_DIM = 64
BATCH = 16384

_info = plsc.get_sparse_core_info()
_NC, _NS = _info.num_cores, _info.num_subcores
_NW = _NC * _NS
_B_PER_W = BATCH // _NW
_CHUNK = 128
_NCHUNK = _B_PER_W // _CHUNK


@functools.partial(
    pl.kernel,
    mesh=plsc.VectorSubcoreMesh(core_axis_name="c", subcore_axis_name="s"),
    out_type=jax.ShapeDtypeStruct((BATCH, EMBED_DIM), jnp.float32),
    scratch_types=[
        pltpu.VMEM((_B_PER_W,), jnp.int32),
        [pltpu.VMEM((_CHUNK, EMBED_DIM), jnp.float32) for _ in range(_NCHUNK)],
        [pltpu.SemaphoreType.DMA for _ in range(_NCHUNK)],
        pltpu.SemaphoreType.DMA,
    ],
    compiler_params=pltpu.CompilerParams(use_tc_tiling_on_sc=True),
)
def _gather_kernel(idx_hbm, table_hbm, out_hbm, idx_v, rows, gsems, ssem):
    wid = lax.axis_index("s") * _NC + lax.axis_index("c")
    base = wid * _B_PER_W
    pltpu.sync_copy(idx_hbm.at[pl.ds(base, _B_PER_W)], idx_v)

    def fire_chunk(c, buf, sem):
        def body(j, _):
            v = idx_v[pl.ds(c * _CHUNK + j * 16, 16)]
            for k in range(16):
                pltpu.async_copy(
                    table_hbm.at[pl.ds(v[k], 1)],
                    buf.at[pl.ds(j * 16 + k, 1)],
                    sem,
                )
            return 0

        lax.fori_loop(0, _CHUNK // 16, body, 0)

    def drain_chunk(buf, sem):
        pltpu.make_async_copy(
            table_hbm.at[pl.ds(0, _CHUNK)], buf, sem
        ).wait()

    for c in range(_NCHUNK):
        fire_chunk(c, rows[c], gsems[c])
    scatters = []
    for c in range(_NCHUNK):
        drain_chunk(rows[c], gsems[c])
        scatters.append(
            pltpu.async_copy(
                rows[c], out_hbm.at[pl.ds(base + c * _CHUNK, _CHUNK)], ssem
            )
        )
    for s in scatters:
        s.wait()


def kernel(x, table):
    return _gather_kernel(x.astype(jnp.int32), table)
